# baseline (speedup 1.0000x reference)
import functools

import jax
import jax.numpy as jnp
from jax.experimental import pallas as pl
from jax.experimental.pallas import tpu as pltpu

_THRESH = 0.6
_P_SLOTS = 6


def _h1_body(feat_ref, w1_ref, h1_ref):
    h1_ref[...] = jnp.dot(feat_ref[...], w1_ref[...],
                          preferred_element_type=jnp.float32)


def _num_phases(nb):
    total = nb * (nb + 1) // 2
    base, phases = 0, 0
    while base < total:
        avail = min(phases * (phases + 1) // 2, total)
        base += min(max(avail - base, 0), _P_SLOTS)
        phases += 1
    return phases


def _fused_body(b2, bc, nb, h1_ref, w2_ref, adj_ref, out_ref,
                hn_ref, y_ref, l1r_ref, l1c_ref):
    c = pl.program_id(0)
    t = pl.program_id(1)
    total = nb * (nb + 1) // 2
    spc = bc // b2

    @pl.when(jnp.logical_and(c == 0, t == 0))
    def _init():
        out_ref[...] = jnp.zeros_like(out_ref)
        l1r_ref[...] = jnp.zeros_like(l1r_ref)
        l1c_ref[...] = jnp.zeros_like(l1c_ref)

    is_s = jnp.logical_and(jnp.logical_and(t % 2 == 0, t <= 2 * spc - 2),
                           c < nb)

    @pl.when(is_s)
    def _stage2():
        sb = spc * c + t // 2
        h = jnp.dot(adj_ref[...], h1_ref[...],
                    preferred_element_type=jnp.float32)
        h = jnp.maximum(h, 0.0)
        norm = jnp.maximum(jnp.sqrt(jnp.sum(h * h, axis=1, keepdims=True)),
                           1e-12)
        hn_ref[pl.ds(sb * b2, b2), :] = h / norm
        y_ref[pl.ds(sb * b2, b2), :] = jnp.dot(
            h, w2_ref[...], preferred_element_type=jnp.float32)

    def _base_step(p, b):
        avail_p = jnp.minimum(p * (p + 1) // 2, total)
        return b + jnp.clip(avail_p - b, 0, _P_SLOTS)

    base = jax.lax.fori_loop(0, c, _base_step, jnp.int32(0))
    avail = jnp.minimum(c * (c + 1) // 2, total)
    n_here = jnp.clip(avail - base, 0, _P_SLOTS)
    in_tail = t >= 2 * spc
    r = jnp.where(in_tail, spc + (t - 2 * spc), t // 2)
    is_p = jnp.logical_or(in_tail, t % 2 == 1)
    q = base + r

    @pl.when(jnp.logical_and(is_p, r < n_here))
    def _pair():
        def _find_j(k, acc):
            return jnp.where(q >= (k + 1) * (k + 2) // 2, k + 1, acc)

        j = jax.lax.fori_loop(0, nb, _find_j, jnp.int32(0))
        i = q - j * (j + 1) // 2
        hni = hn_ref[pl.ds(i * bc, bc), :]
        hnj = hn_ref[pl.ds(j * bc, bc), :]
        sim = jax.lax.dot_general(hni, hnj, (((1,), (1,)), ((), ())),
                                  preferred_element_type=jnp.float32)
        da = jnp.where(sim >= _THRESH, sim, 0.0)
        yj = y_ref[pl.ds(j * bc, bc), :]
        out_ref[pl.ds(i * bc, bc), :] += jax.lax.dot_general(
            da, yj, (((1,), (0,)), ((), ())),
            preferred_element_type=jnp.float32)
        l1r_ref[pl.ds(i * bc, bc), :] += jnp.sum(da, axis=1, keepdims=True)

        @pl.when(j > i)
        def _mirror():
            yi = y_ref[pl.ds(i * bc, bc), :]
            out_ref[pl.ds(j * bc, bc), :] += jax.lax.dot_general(
                da, yi, (((0,), (0,)), ((), ())),
                preferred_element_type=jnp.float32)
            l1c_ref[j, 0:1, :] += jnp.sum(da, axis=0, keepdims=True)

        @pl.when(q >= (nb - 1) * nb // 2)
        def _finalize():
            fi = q - (nb - 1) * nb // 2
            l1c_t = jax.lax.dot_general(
                l1c_ref[fi, 0:1, :], jnp.ones((1, 1), jnp.float32),
                (((0,), (0,)), ((), ())),
                preferred_element_type=jnp.float32)
            l1 = jnp.maximum(l1r_ref[pl.ds(fi * bc, bc), :] + l1c_t, 1e-12)
            out_ref[pl.ds(fi * bc, bc), :] = (
                out_ref[pl.ds(fi * bc, bc), :] / l1)


def _pick_block(n, candidates):
    for c in candidates:
        if n % c == 0:
            return c
    return n


def kernel(feat, adj_spatial, W1, W2):
    n, in_feat = feat.shape
    hid = W1.shape[1]
    out_feat = W2.shape[1]
    f32 = jnp.float32

    b1 = _pick_block(n, [2000, 1000, 400, 200, 8])
    h1 = pl.pallas_call(
        _h1_body,
        grid=(n // b1,),
        in_specs=[
            pl.BlockSpec((b1, in_feat), lambda i: (i, 0)),
            pl.BlockSpec((in_feat, hid), lambda i: (0, 0)),
        ],
        out_specs=pl.BlockSpec((b1, hid), lambda i: (i, 0)),
        out_shape=jax.ShapeDtypeStruct((n, hid), f32),
    )(feat, W1)

    bc = _pick_block(n, [1000, 400, 200, 8])
    b2 = _pick_block(bc, [200, 100, 8])
    nb = n // bc
    spc = bc // b2
    n_s = n // b2
    phases = max(_num_phases(nb), nb)
    slots = spc + _P_SLOTS

    def _adj_idx(c, t):
        return (jnp.minimum(spc * c + jnp.minimum(t // 2, spc - 1), n_s - 1),
                0)

    out = pl.pallas_call(
        functools.partial(_fused_body, b2, bc, nb),
        grid=(phases, slots),
        in_specs=[
            pl.BlockSpec((n, hid), lambda c, t: (0, 0)),
            pl.BlockSpec((hid, out_feat), lambda c, t: (0, 0)),
            pl.BlockSpec((b2, n), _adj_idx),
        ],
        out_specs=pl.BlockSpec((n, out_feat), lambda c, t: (0, 0)),
        out_shape=jax.ShapeDtypeStruct((n, out_feat), f32),
        scratch_shapes=[
            pltpu.VMEM((n, hid), f32),
            pltpu.VMEM((n, out_feat), f32),
            pltpu.VMEM((n, 1), f32),
            pltpu.VMEM((nb, 8, bc), f32),
        ],
    )(h1, W2, adj_spatial)

    return out


# speedup vs baseline: 1.3336x; 1.3336x over previous
import jax
import jax.numpy as jnp
from jax.experimental import pallas as pl
from jax.experimental.pallas import tpu as pltpu

_THRESH = 0.6


def _h_body(feat_ref, w1_ref, adj_ref, w2_ref, hn_ref, y_ref, h1_ref):
    @pl.when(pl.program_id(0) == 0)
    def _prologue():
        h1_ref[...] = jnp.dot(feat_ref[...], w1_ref[...],
                              preferred_element_type=jnp.float32)

    h = jnp.dot(adj_ref[...], h1_ref[...], preferred_element_type=jnp.float32)
    h = jnp.maximum(h, 0.0)
    norm = jnp.maximum(jnp.sqrt(jnp.sum(h * h, axis=1, keepdims=True)), 1e-12)
    hn_ref[...] = (h / norm).astype(jnp.bfloat16)
    y_ref[...] = jnp.dot(h, w2_ref[...],
                         preferred_element_type=jnp.float32).astype(jnp.bfloat16)


def _agg_body(hni_ref, hn_ref, y_ref, out_ref):
    sim = jax.lax.dot_general(hni_ref[...], hn_ref[...],
                              (((1,), (1,)), ((), ())),
                              preferred_element_type=jnp.float32)
    da = jnp.where(sim >= _THRESH, sim, 0.0)
    l1 = jnp.maximum(jnp.sum(da, axis=1, keepdims=True), 1e-12)
    acc = jax.lax.dot_general(da.astype(jnp.bfloat16), y_ref[...],
                              (((1,), (0,)), ((), ())),
                              preferred_element_type=jnp.float32)
    out_ref[...] = acc / l1


def _pick_block(n, candidates):
    for c in candidates:
        if n % c == 0:
            return c
    return n


def kernel(feat, adj_spatial, W1, W2):
    n, in_feat = feat.shape
    hid = W1.shape[1]
    out_feat = W2.shape[1]
    f32 = jnp.float32

    b2 = _pick_block(n, [400, 200, 8])
    hn, y = pl.pallas_call(
        _h_body,
        grid=(n // b2,),
        in_specs=[
            pl.BlockSpec((n, in_feat), lambda i: (0, 0)),
            pl.BlockSpec((in_feat, hid), lambda i: (0, 0)),
            pl.BlockSpec((b2, n), lambda i: (i, 0)),
            pl.BlockSpec((hid, out_feat), lambda i: (0, 0)),
        ],
        out_specs=[
            pl.BlockSpec((b2, hid), lambda i: (i, 0)),
            pl.BlockSpec((b2, out_feat), lambda i: (i, 0)),
        ],
        out_shape=[
            jax.ShapeDtypeStruct((n, hid), jnp.bfloat16),
            jax.ShapeDtypeStruct((n, out_feat), jnp.bfloat16),
        ],
        scratch_shapes=[pltpu.VMEM((n, hid), f32)],
    )(feat, W1, adj_spatial, W2)

    b3 = _pick_block(n, [1000, 400, 200, 8])
    out = pl.pallas_call(
        _agg_body,
        grid=(n // b3,),
        in_specs=[
            pl.BlockSpec((b3, hid), lambda i: (i, 0)),
            pl.BlockSpec((n, hid), lambda i: (0, 0)),
            pl.BlockSpec((n, out_feat), lambda i: (0, 0)),
        ],
        out_specs=pl.BlockSpec((b3, out_feat), lambda i: (i, 0)),
        out_shape=jax.ShapeDtypeStruct((n, out_feat), f32),
    )(hn, hn, y)

    return out
